# baseline (speedup 1.0000x reference)
import functools

import jax
import jax.numpy as jnp
from jax import lax
from jax.experimental import pallas as pl
from jax.experimental.pallas import tpu as pltpu
from jax.experimental.pallas import tpu_sc as plsc

_VOXEL_XY = 0.05
_BEV_STRIDE = 8
_PC_X0 = 0.0
_PC_Y0 = -40.0

_B, _K = 4, 2048
_N = _B * _K
_CIN, _COUT = 256, 128
_H = _W = 188
_HW = _H * _W

_NC, _NS, _LANES = 2, 16, 16
_NWORK = _NC * _NS
_PTS_PER_W = _N // _NWORK
_CH = 16
_NCHUNK = _PTS_PER_W // _CH


def _prep(xv, yv, b, c):
    xs = xv[pl.ds(c * _CH, _CH)]
    ys = yv[pl.ds(c * _CH, _CH)]
    x0 = xs.astype(jnp.int32)
    y0 = ys.astype(jnp.int32)
    x0c = jnp.minimum(jnp.maximum(x0, 0), _W - 1)
    x1c = jnp.minimum(jnp.maximum(x0 + 1, 0), _W - 1)
    y0c = jnp.minimum(jnp.maximum(y0, 0), _H - 1)
    y1c = jnp.minimum(jnp.maximum(y0 + 1, 0), _H - 1)
    x0f = x0c.astype(jnp.float32)
    x1f = x1c.astype(jnp.float32)
    y0f = y0c.astype(jnp.float32)
    y1f = y1c.astype(jnp.float32)
    ra = (y0c * _W + x0c) * 8 + b
    rb = (y1c * _W + x0c) * 8 + b
    rc = (y0c * _W + x1c) * 8 + b
    rd = (y1c * _W + x1c) * 8 + b
    wa = (x1f - xs) * (y1f - ys)
    wb = (x1f - xs) * (ys - y0f)
    wc = (xs - x0f) * (y1f - ys)
    wd = (xs - x0f) * (ys - y0f)
    return (ra, rb, rc, rd), (wa, wb, wc, wd)


def _store_idx(idxv, r4):
    ra, rb, rc, rd = r4
    idxv[pl.ds(0, 16)] = ra
    idxv[pl.ds(16, 16)] = rb
    idxv[pl.ds(32, 16)] = rc
    idxv[pl.ds(48, 16)] = rd
    idxv[pl.ds(64, 16)] = ra + 4
    idxv[pl.ds(80, 16)] = rb + 4
    idxv[pl.ds(96, 16)] = rc + 4
    idxv[pl.ds(112, 16)] = rd + 4


def _combine(rows, w4, lo, hi):
    wa, wb, wc, wd = w4
    for p in range(_CH):
        wa_p = wa[p]
        wb_p = wb[p]
        wc_p = wc[p]
        wd_p = wd[p]
        for s in range(128 // _LANES):
            col = pl.ds(s * _LANES, _LANES)
            lo[p, col] = (wa_p * rows[p, col] + wb_p * rows[16 + p, col]
                          + wc_p * rows[32 + p, col]
                          + wd_p * rows[48 + p, col])
            hi[p, col] = (wa_p * rows[64 + p, col]
                          + wb_p * rows[80 + p, col]
                          + wc_p * rows[96 + p, col]
                          + wd_p * rows[112 + p, col])


def _sc_body(q_hbm, x_hbm, y_hbm, lo_hbm, hi_hbm, xv, yv, idx0, idx1,
             rows0, rows1, lo0, hi0, lo1, hi1, sem0, sem1, wsem0, wsem1):
    wid = lax.axis_index("s") * _NC + lax.axis_index("c")
    base = wid * _PTS_PER_W
    b = base // _K
    pltpu.sync_copy(x_hbm.at[pl.ds(base, _PTS_PER_W)], xv)
    pltpu.sync_copy(y_hbm.at[pl.ds(base, _PTS_PER_W)], yv)

    r4, w4 = _prep(xv, yv, b, 0)
    _store_idx(idx0, r4)
    pltpu.async_copy(q_hbm.at[idx0], rows0, sem0)

    def pair(g, w_cur):
        c0 = 2 * g
        r1, w1 = _prep(xv, yv, b, c0 + 1)
        _store_idx(idx1, r1)
        pltpu.async_copy(q_hbm.at[idx1], rows1, sem1)
        pltpu.make_async_copy(q_hbm.at[idx0], rows0, sem0).wait()

        @pl.when(g > 0)
        def _():
            pltpu.make_async_copy(lo0, lo_hbm.at[pl.ds(base, _CH)], wsem0).wait()
            pltpu.make_async_copy(hi0, hi_hbm.at[pl.ds(base, _CH)], wsem0).wait()

        _combine(rows0, w_cur, lo0, hi0)
        pltpu.async_copy(lo0, lo_hbm.at[pl.ds(base + c0 * _CH, _CH)], wsem0)
        pltpu.async_copy(hi0, hi_hbm.at[pl.ds(base + c0 * _CH, _CH)], wsem0)
        r2, w2 = _prep(xv, yv, b, jnp.minimum(c0 + 2, _NCHUNK - 1))

        @pl.when(g < _NCHUNK // 2 - 1)
        def _():
            _store_idx(idx0, r2)
            pltpu.async_copy(q_hbm.at[idx0], rows0, sem0)

        pltpu.make_async_copy(q_hbm.at[idx1], rows1, sem1).wait()

        @pl.when(g > 0)
        def _():
            pltpu.make_async_copy(lo1, lo_hbm.at[pl.ds(base, _CH)], wsem1).wait()
            pltpu.make_async_copy(hi1, hi_hbm.at[pl.ds(base, _CH)], wsem1).wait()

        _combine(rows1, w1, lo1, hi1)
        pltpu.async_copy(lo1, lo_hbm.at[pl.ds(base + (c0 + 1) * _CH, _CH)], wsem1)
        pltpu.async_copy(hi1, hi_hbm.at[pl.ds(base + (c0 + 1) * _CH, _CH)], wsem1)
        return w2

    lax.fori_loop(0, _NCHUNK // 2, pair, w4)
    pltpu.make_async_copy(lo0, lo_hbm.at[pl.ds(base, _CH)], wsem0).wait()
    pltpu.make_async_copy(hi0, hi_hbm.at[pl.ds(base, _CH)], wsem0).wait()
    pltpu.make_async_copy(lo1, lo_hbm.at[pl.ds(base, _CH)], wsem1).wait()
    pltpu.make_async_copy(hi1, hi_hbm.at[pl.ds(base, _CH)], wsem1).wait()


@functools.cache
def _sc_gather():
    return pl.kernel(
        _sc_body,
        out_type=(jax.ShapeDtypeStruct((_N, 128), jnp.float32),
                  jax.ShapeDtypeStruct((_N, 128), jnp.float32)),
        mesh=plsc.VectorSubcoreMesh(core_axis_name="c", subcore_axis_name="s",
                                    num_cores=_NC, num_subcores=_NS),
        scratch_types=[
            pltpu.VMEM((_PTS_PER_W,), jnp.float32),
            pltpu.VMEM((_PTS_PER_W,), jnp.float32),
            pltpu.VMEM((8 * _CH,), jnp.int32),
            pltpu.VMEM((8 * _CH,), jnp.int32),
            pltpu.VMEM((8 * _CH, 128), jnp.float32),
            pltpu.VMEM((8 * _CH, 128), jnp.float32),
            pltpu.VMEM((_CH, 128), jnp.float32),
            pltpu.VMEM((_CH, 128), jnp.float32),
            pltpu.VMEM((_CH, 128), jnp.float32),
            pltpu.VMEM((_CH, 128), jnp.float32),
            pltpu.SemaphoreType.DMA,
            pltpu.SemaphoreType.DMA,
            pltpu.SemaphoreType.DMA,
            pltpu.SemaphoreType.DMA,
        ],
    )


def _mm_bn_body(lo_ref, hi_ref, w_ref, g_ref, b_ref, out_ref):
    w = w_ref[...]
    h = (lax.dot_general(lo_ref[...], w[:, :128], (((1,), (1,)), ((), ())),
                         preferred_element_type=jnp.float32)
         + lax.dot_general(hi_ref[...], w[:, 128:], (((1,), (1,)), ((), ())),
                           preferred_element_type=jnp.float32))
    m = jnp.mean(h, axis=0, keepdims=True)
    d = h - m
    v = jnp.mean(d * d, axis=0, keepdims=True)
    out_ref[...] = jnp.maximum(
        d * lax.rsqrt(v + 1e-5) * g_ref[...] + b_ref[...], 0.0)


def _mm_bn(lo, hi, w_fuse, gamma, beta):
    return pl.pallas_call(
        _mm_bn_body,
        out_shape=jax.ShapeDtypeStruct((_N, _COUT), jnp.float32),
    )(lo, hi, w_fuse, gamma.reshape(1, _COUT), beta.reshape(1, _COUT))


def kernel(keypoints, spatial_features, W_fuse, bn_gamma, bn_beta):
    x_idxs = ((keypoints[:, :, 0] - _PC_X0) / _VOXEL_XY / _BEV_STRIDE).reshape(_N)
    y_idxs = ((keypoints[:, :, 1] - _PC_Y0) / _VOXEL_XY / _BEV_STRIDE).reshape(_N)
    q = spatial_features.transpose(2, 3, 0, 1).reshape(_HW, _B, 2, 128)
    q = q.transpose(0, 2, 1, 3).reshape(_HW * 8, 128)
    lo, hi = _sc_gather()(q, x_idxs, y_idxs)
    return _mm_bn(lo, hi, W_fuse, bn_gamma, bn_beta)


# speedup vs baseline: 1.0199x; 1.0199x over previous
import functools

import jax
import jax.numpy as jnp
from jax import lax
from jax.experimental import pallas as pl
from jax.experimental.pallas import tpu as pltpu
from jax.experimental.pallas import tpu_sc as plsc

_VOXEL_XY = 0.05
_BEV_STRIDE = 8
_PC_X0 = 0.0
_PC_Y0 = -40.0

_B, _K = 4, 2048
_N = _B * _K
_CIN, _COUT = 256, 128
_H = _W = 188
_HW = _H * _W

_NC, _NS, _LANES = 2, 16, 16
_NWORK = _NC * _NS
_PTS_PER_W = _N // _NWORK
_CH = 16
_NCHUNK = _PTS_PER_W // _CH


def _prep(xv, yv, b, c):
    xs = xv[pl.ds(c * _CH, _CH)]
    ys = yv[pl.ds(c * _CH, _CH)]
    x0 = xs.astype(jnp.int32)
    y0 = ys.astype(jnp.int32)
    x0c = jnp.minimum(jnp.maximum(x0, 0), _W - 1)
    x1c = jnp.minimum(jnp.maximum(x0 + 1, 0), _W - 1)
    y0c = jnp.minimum(jnp.maximum(y0, 0), _H - 1)
    y1c = jnp.minimum(jnp.maximum(y0 + 1, 0), _H - 1)
    x0f = x0c.astype(jnp.float32)
    x1f = x1c.astype(jnp.float32)
    y0f = y0c.astype(jnp.float32)
    y1f = y1c.astype(jnp.float32)
    ra = (y0c * _W + x0c) * 8 + b
    rb = (y1c * _W + x0c) * 8 + b
    rc = (y0c * _W + x1c) * 8 + b
    rd = (y1c * _W + x1c) * 8 + b
    wa = (x1f - xs) * (y1f - ys)
    wb = (x1f - xs) * (ys - y0f)
    wc = (xs - x0f) * (y1f - ys)
    wd = (xs - x0f) * (ys - y0f)
    return (ra, rb, rc, rd), (wa, wb, wc, wd)


def _store_idx(idxv, r4):
    ra, rb, rc, rd = r4
    idxv[pl.ds(0, 16)] = ra
    idxv[pl.ds(16, 16)] = rb
    idxv[pl.ds(32, 16)] = rc
    idxv[pl.ds(48, 16)] = rd
    idxv[pl.ds(64, 16)] = ra + 4
    idxv[pl.ds(80, 16)] = rb + 4
    idxv[pl.ds(96, 16)] = rc + 4
    idxv[pl.ds(112, 16)] = rd + 4


def _combine(rows, w4, lo, hi):
    wa, wb, wc, wd = w4
    for p in range(_CH):
        wa_p = wa[p]
        wb_p = wb[p]
        wc_p = wc[p]
        wd_p = wd[p]
        for s in range(128 // _LANES):
            col = pl.ds(s * _LANES, _LANES)
            lo[p, col] = (wa_p * rows[p, col] + wb_p * rows[16 + p, col]
                          + wc_p * rows[32 + p, col]
                          + wd_p * rows[48 + p, col])
            hi[p, col] = (wa_p * rows[64 + p, col]
                          + wb_p * rows[80 + p, col]
                          + wc_p * rows[96 + p, col]
                          + wd_p * rows[112 + p, col])


def _sc_body(q_hbm, x_hbm, y_hbm, lo_hbm, hi_hbm, xv, yv, idx0, idx1,
             rows0, rows1, lo0, hi0, lo1, hi1, sem0, sem1):
    wid = lax.axis_index("s") * _NC + lax.axis_index("c")
    base = wid * _PTS_PER_W
    b = base // _K
    pltpu.sync_copy(x_hbm.at[pl.ds(base, _PTS_PER_W)], xv)
    pltpu.sync_copy(y_hbm.at[pl.ds(base, _PTS_PER_W)], yv)

    r4, w4 = _prep(xv, yv, b, 0)
    _store_idx(idx0, r4)
    pltpu.async_copy(q_hbm.at[idx0], rows0, sem0)

    def pair(g, w_cur):
        c0 = 2 * g
        r1, w1 = _prep(xv, yv, b, c0 + 1)
        _store_idx(idx1, r1)
        pltpu.async_copy(q_hbm.at[idx1], rows1, sem1)
        pltpu.make_async_copy(q_hbm.at[idx0], rows0, sem0).wait()
        _combine(rows0, w_cur, lo0, hi0)
        pltpu.sync_copy(lo0, lo_hbm.at[pl.ds(base + c0 * _CH, _CH)])
        pltpu.sync_copy(hi0, hi_hbm.at[pl.ds(base + c0 * _CH, _CH)])
        r2, w2 = _prep(xv, yv, b, jnp.minimum(c0 + 2, _NCHUNK - 1))

        @pl.when(g < _NCHUNK // 2 - 1)
        def _():
            _store_idx(idx0, r2)
            pltpu.async_copy(q_hbm.at[idx0], rows0, sem0)

        pltpu.make_async_copy(q_hbm.at[idx1], rows1, sem1).wait()
        _combine(rows1, w1, lo1, hi1)
        pltpu.sync_copy(lo1, lo_hbm.at[pl.ds(base + (c0 + 1) * _CH, _CH)])
        pltpu.sync_copy(hi1, hi_hbm.at[pl.ds(base + (c0 + 1) * _CH, _CH)])
        return w2

    lax.fori_loop(0, _NCHUNK // 2, pair, w4)


@functools.cache
def _sc_gather():
    return pl.kernel(
        _sc_body,
        out_type=(jax.ShapeDtypeStruct((_N, 128), jnp.float32),
                  jax.ShapeDtypeStruct((_N, 128), jnp.float32)),
        mesh=plsc.VectorSubcoreMesh(core_axis_name="c", subcore_axis_name="s",
                                    num_cores=_NC, num_subcores=_NS),
        scratch_types=[
            pltpu.VMEM((_PTS_PER_W,), jnp.float32),
            pltpu.VMEM((_PTS_PER_W,), jnp.float32),
            pltpu.VMEM((8 * _CH,), jnp.int32),
            pltpu.VMEM((8 * _CH,), jnp.int32),
            pltpu.VMEM((8 * _CH, 128), jnp.float32),
            pltpu.VMEM((8 * _CH, 128), jnp.float32),
            pltpu.VMEM((_CH, 128), jnp.float32),
            pltpu.VMEM((_CH, 128), jnp.float32),
            pltpu.VMEM((_CH, 128), jnp.float32),
            pltpu.VMEM((_CH, 128), jnp.float32),
            pltpu.SemaphoreType.DMA,
            pltpu.SemaphoreType.DMA,
        ],
    )


def _mm_bn_body(lo_ref, hi_ref, w_ref, g_ref, b_ref, out_ref):
    w = w_ref[...].astype(jnp.bfloat16)
    h = (lax.dot_general(lo_ref[...].astype(jnp.bfloat16), w[:, :128],
                         (((1,), (1,)), ((), ())),
                         preferred_element_type=jnp.float32)
         + lax.dot_general(hi_ref[...].astype(jnp.bfloat16), w[:, 128:],
                           (((1,), (1,)), ((), ())),
                           preferred_element_type=jnp.float32))
    m = jnp.mean(h, axis=0, keepdims=True)
    d = h - m
    v = jnp.mean(d * d, axis=0, keepdims=True)
    out_ref[...] = jnp.maximum(
        d * lax.rsqrt(v + 1e-5) * g_ref[...] + b_ref[...], 0.0)


def _mm_bn(lo, hi, w_fuse, gamma, beta):
    return pl.pallas_call(
        _mm_bn_body,
        out_shape=jax.ShapeDtypeStruct((_N, _COUT), jnp.float32),
    )(lo, hi, w_fuse, gamma.reshape(1, _COUT), beta.reshape(1, _COUT))


def kernel(keypoints, spatial_features, W_fuse, bn_gamma, bn_beta):
    x_idxs = ((keypoints[:, :, 0] - _PC_X0) / _VOXEL_XY / _BEV_STRIDE).reshape(_N)
    y_idxs = ((keypoints[:, :, 1] - _PC_Y0) / _VOXEL_XY / _BEV_STRIDE).reshape(_N)
    q = spatial_features.transpose(2, 3, 0, 1).reshape(_HW, _B, 2, 128)
    q = q.transpose(0, 2, 1, 3).reshape(_HW * 8, 128)
    lo, hi = _sc_gather()(q, x_idxs, y_idxs)
    return _mm_bn(lo, hi, W_fuse, bn_gamma, bn_beta)


# speedup vs baseline: 1.0213x; 1.0013x over previous
import functools

import jax
import jax.numpy as jnp
from jax import lax
from jax.experimental import pallas as pl
from jax.experimental.pallas import tpu as pltpu
from jax.experimental.pallas import tpu_sc as plsc

_VOXEL_XY = 0.05
_BEV_STRIDE = 8
_PC_X0 = 0.0
_PC_Y0 = -40.0

_B, _K = 4, 2048
_N = _B * _K
_CIN, _COUT = 256, 128
_H = _W = 188
_HW = _H * _W

_NC, _NS, _LANES = 2, 16, 16
_NWORK = _NC * _NS
_PTS_PER_W = _N // _NWORK
_CH = 16
_NCHUNK = _PTS_PER_W // _CH


def _prep(xv, yv, b, c):
    xs = xv[pl.ds(c * _CH, _CH)]
    ys = yv[pl.ds(c * _CH, _CH)]
    x0 = xs.astype(jnp.int32)
    y0 = ys.astype(jnp.int32)
    x0c = jnp.minimum(jnp.maximum(x0, 0), _W - 1)
    x1c = jnp.minimum(jnp.maximum(x0 + 1, 0), _W - 1)
    y0c = jnp.minimum(jnp.maximum(y0, 0), _H - 1)
    y1c = jnp.minimum(jnp.maximum(y0 + 1, 0), _H - 1)
    x0f = x0c.astype(jnp.float32)
    x1f = x1c.astype(jnp.float32)
    y0f = y0c.astype(jnp.float32)
    y1f = y1c.astype(jnp.float32)
    ra = (y0c * _W + x0c) * 8 + b
    rb = (y1c * _W + x0c) * 8 + b
    rc = (y0c * _W + x1c) * 8 + b
    rd = (y1c * _W + x1c) * 8 + b
    wa = (x1f - xs) * (y1f - ys)
    wb = (x1f - xs) * (ys - y0f)
    wc = (xs - x0f) * (y1f - ys)
    wd = (xs - x0f) * (ys - y0f)
    return (ra, rb, rc, rd), (wa, wb, wc, wd)


def _store_idx(idxv, r4):
    ra, rb, rc, rd = r4
    idxv[pl.ds(0, 16)] = ra
    idxv[pl.ds(16, 16)] = rb
    idxv[pl.ds(32, 16)] = rc
    idxv[pl.ds(48, 16)] = rd
    idxv[pl.ds(64, 16)] = ra + 4
    idxv[pl.ds(80, 16)] = rb + 4
    idxv[pl.ds(96, 16)] = rc + 4
    idxv[pl.ds(112, 16)] = rd + 4


def _combine(rows, w4, lo, hi):
    wa, wb, wc, wd = w4
    for p in range(_CH):
        wa_p = wa[p]
        wb_p = wb[p]
        wc_p = wc[p]
        wd_p = wd[p]
        for s in range(128 // _LANES):
            col = pl.ds(s * _LANES, _LANES)
            lo[p, col] = (wa_p * rows[p, col] + wb_p * rows[16 + p, col]
                          + wc_p * rows[32 + p, col]
                          + wd_p * rows[48 + p, col])
            hi[p, col] = (wa_p * rows[64 + p, col]
                          + wb_p * rows[80 + p, col]
                          + wc_p * rows[96 + p, col]
                          + wd_p * rows[112 + p, col])


def _sc_body(q_hbm, x_hbm, y_hbm, lo_hbm, hi_hbm, xv, yv, idx0, idx1,
             rows0, rows1, lo0, hi0, lo1, hi1, sem0, sem1):
    wid = lax.axis_index("s") * _NC + lax.axis_index("c")
    base = wid * _PTS_PER_W
    b = base // _K
    pltpu.sync_copy(x_hbm.at[pl.ds(base, _PTS_PER_W)], xv)
    pltpu.sync_copy(y_hbm.at[pl.ds(base, _PTS_PER_W)], yv)

    def gather(idxv, rows, sem):
        pltpu.async_copy(q_hbm.at[idxv.at[pl.ds(0, 64)]],
                         rows.at[pl.ds(0, 64)], sem)
        pltpu.async_copy(q_hbm.at[idxv.at[pl.ds(64, 64)]],
                         rows.at[pl.ds(64, 64)], sem)

    def wait(idxv, rows, sem):
        pltpu.make_async_copy(q_hbm.at[idxv.at[pl.ds(0, 64)]],
                              rows.at[pl.ds(0, 64)], sem).wait()
        pltpu.make_async_copy(q_hbm.at[idxv.at[pl.ds(64, 64)]],
                              rows.at[pl.ds(64, 64)], sem).wait()

    r4, w4 = _prep(xv, yv, b, 0)
    _store_idx(idx0, r4)
    gather(idx0, rows0, sem0)

    def pair(g, w_cur):
        c0 = 2 * g
        r1, w1 = _prep(xv, yv, b, c0 + 1)
        _store_idx(idx1, r1)
        gather(idx1, rows1, sem1)
        wait(idx0, rows0, sem0)
        _combine(rows0, w_cur, lo0, hi0)
        pltpu.sync_copy(lo0, lo_hbm.at[pl.ds(base + c0 * _CH, _CH)])
        pltpu.sync_copy(hi0, hi_hbm.at[pl.ds(base + c0 * _CH, _CH)])
        r2, w2 = _prep(xv, yv, b, jnp.minimum(c0 + 2, _NCHUNK - 1))

        @pl.when(g < _NCHUNK // 2 - 1)
        def _():
            _store_idx(idx0, r2)
            gather(idx0, rows0, sem0)

        wait(idx1, rows1, sem1)
        _combine(rows1, w1, lo1, hi1)
        pltpu.sync_copy(lo1, lo_hbm.at[pl.ds(base + (c0 + 1) * _CH, _CH)])
        pltpu.sync_copy(hi1, hi_hbm.at[pl.ds(base + (c0 + 1) * _CH, _CH)])
        return w2

    lax.fori_loop(0, _NCHUNK // 2, pair, w4)


@functools.cache
def _sc_gather():
    return pl.kernel(
        _sc_body,
        out_type=(jax.ShapeDtypeStruct((_N, 128), jnp.float32),
                  jax.ShapeDtypeStruct((_N, 128), jnp.float32)),
        mesh=plsc.VectorSubcoreMesh(core_axis_name="c", subcore_axis_name="s",
                                    num_cores=_NC, num_subcores=_NS),
        scratch_types=[
            pltpu.VMEM((_PTS_PER_W,), jnp.float32),
            pltpu.VMEM((_PTS_PER_W,), jnp.float32),
            pltpu.VMEM((8 * _CH,), jnp.int32),
            pltpu.VMEM((8 * _CH,), jnp.int32),
            pltpu.VMEM((8 * _CH, 128), jnp.float32),
            pltpu.VMEM((8 * _CH, 128), jnp.float32),
            pltpu.VMEM((_CH, 128), jnp.float32),
            pltpu.VMEM((_CH, 128), jnp.float32),
            pltpu.VMEM((_CH, 128), jnp.float32),
            pltpu.VMEM((_CH, 128), jnp.float32),
            pltpu.SemaphoreType.DMA,
            pltpu.SemaphoreType.DMA,
        ],
    )


def _mm_bn_body(lo_ref, hi_ref, w_ref, g_ref, b_ref, out_ref):
    w = w_ref[...].astype(jnp.bfloat16)
    h = (lax.dot_general(lo_ref[...].astype(jnp.bfloat16), w[:, :128],
                         (((1,), (1,)), ((), ())),
                         preferred_element_type=jnp.float32)
         + lax.dot_general(hi_ref[...].astype(jnp.bfloat16), w[:, 128:],
                           (((1,), (1,)), ((), ())),
                           preferred_element_type=jnp.float32))
    m = jnp.mean(h, axis=0, keepdims=True)
    d = h - m
    v = jnp.mean(d * d, axis=0, keepdims=True)
    out_ref[...] = jnp.maximum(
        d * lax.rsqrt(v + 1e-5) * g_ref[...] + b_ref[...], 0.0)


def _mm_bn(lo, hi, w_fuse, gamma, beta):
    return pl.pallas_call(
        _mm_bn_body,
        out_shape=jax.ShapeDtypeStruct((_N, _COUT), jnp.float32),
    )(lo, hi, w_fuse, gamma.reshape(1, _COUT), beta.reshape(1, _COUT))


def kernel(keypoints, spatial_features, W_fuse, bn_gamma, bn_beta):
    x_idxs = ((keypoints[:, :, 0] - _PC_X0) / _VOXEL_XY / _BEV_STRIDE).reshape(_N)
    y_idxs = ((keypoints[:, :, 1] - _PC_Y0) / _VOXEL_XY / _BEV_STRIDE).reshape(_N)
    q = spatial_features.transpose(2, 3, 0, 1).reshape(_HW, _B, 2, 128)
    q = q.transpose(0, 2, 1, 3).reshape(_HW * 8, 128)
    lo, hi = _sc_gather()(q, x_idxs, y_idxs)
    return _mm_bn(lo, hi, W_fuse, bn_gamma, bn_beta)
